# baseline (speedup 1.0000x reference)
import functools

import jax
import jax.numpy as jnp
from jax import lax
from jax.experimental import pallas as pl
from jax.experimental.pallas import tpu as pltpu
from jax.experimental.pallas import tpu_sc as plsc

NC = 2
NS = 16
CK = 128
CPT = 80


def _zero_f32(ref, rows, cols):
    z = jnp.zeros((16,), jnp.float32)

    def body(i, _):
        for g in range(cols // 16):
            ref[i, pl.ds(g * 16, 16)] = z
        return 0

    lax.fori_loop(0, rows, body, 0)


def _make_degree_kernel(n):
    npad = ((n + NS * 16 - 1) // (NS * 16)) * (NS * 16)
    rpt = npad // NS
    mesh = plsc.VectorSubcoreMesh(
        core_axis_name="c", subcore_axis_name="s", num_cores=NC,
        num_subcores=NS)

    def body(dstm_hbm, deg_out, deg_sp, zbuf, ones_v, idx_all):
        c = lax.axis_index("c")
        s = lax.axis_index("s")
        w = c * NS + s
        zv = jnp.zeros((16,), jnp.float32)
        ov = jnp.ones((16,), jnp.float32)

        def z16(i, _):
            zbuf[pl.ds(i * 16, 16)] = zv
            return 0

        lax.fori_loop(0, rpt // 16, z16, 0)
        for g in range(CK // 16):
            ones_v[pl.ds(g * 16, 16)] = ov
        pltpu.sync_copy(dstm_hbm.at[pl.ds(w * CPT, CPT), :], idx_all)
        pltpu.sync_copy(zbuf, deg_sp.at[pl.ds(s * rpt, rpt)])
        plsc.subcore_barrier()

        def chunk(j, _):
            pltpu.sync_copy(ones_v, deg_sp.at[idx_all.at[j]], add=True)
            return 0

        lax.fori_loop(0, CPT, chunk, 0)
        plsc.subcore_barrier()

        low = s * rpt
        size_last = n - rpt * (NS - 1)

        @pl.when(s < NS - 1)
        def _():
            pltpu.sync_copy(deg_sp.at[pl.ds(low, rpt)], zbuf)
            pltpu.sync_copy(zbuf, deg_out.at[pl.ds(c * n + low, rpt)])

        @pl.when(s == NS - 1)
        def _():
            pltpu.sync_copy(deg_sp.at[pl.ds(low, size_last)],
                            zbuf.at[pl.ds(0, size_last)])
            pltpu.sync_copy(zbuf.at[pl.ds(0, size_last)],
                            deg_out.at[pl.ds(c * n + low, size_last)])

    return pl.kernel(
        body,
        out_type=jax.ShapeDtypeStruct((NC * n,), jnp.float32),
        mesh=mesh,
        scratch_types=[
            pltpu.VMEM_SHARED((npad,), jnp.float32),
            pltpu.VMEM((rpt,), jnp.float32),
            pltpu.VMEM((CK,), jnp.float32),
            pltpu.VMEM((CPT, CK), jnp.int32),
        ],
    )


def _make_agg_kernel(n, f, tc_tiling=True):
    coc = 80
    rpt = ((n + NS * coc - 1) // (NS * coc)) * coc
    last = n - rpt * (NS - 1)
    assert rpt % coc == 0 and last % coc == 0 and last > 0
    mesh = plsc.VectorSubcoreMesh(
        core_axis_name="c", subcore_axis_name="s", num_cores=NC,
        num_subcores=NS)

    cpt2 = CPT // 2

    def body(y_hbm, srcm_hbm, dstm_hbm, out_hbm, acc_sp,
             rows_a, rows_b, sidx_h, didx_h, sem_a, sem_b):
        c = lax.axis_index("c")
        s = lax.axis_index("s")
        w = c * NS + s
        nch = jnp.where(s == NS - 1, last // coc, rpt // coc)
        nzch = jnp.where(s == NS - 1, (last + coc) // coc, rpt // coc)

        pltpu.async_copy(srcm_hbm.at[pl.ds(w * CPT, cpt2), :], sidx_h, sem_a)
        pltpu.async_copy(dstm_hbm.at[pl.ds(w * CPT, cpt2), :], didx_h, sem_b)

        _zero_f32(rows_a, coc, f)

        def zcp(k, _):
            pltpu.sync_copy(rows_a.at[pl.ds(0, coc), :],
                            acc_sp.at[pl.ds(s * rpt + k * coc, coc), :])
            return 0

        lax.fori_loop(0, nzch, zcp, 0)
        plsc.subcore_barrier()

        def start(jj, rows, sem):
            pltpu.async_copy(y_hbm.at[sidx_h.at[jj]], rows, sem)

        def finish(jj, rows, sem):
            pltpu.make_async_copy(y_hbm.at[sidx_h.at[jj]], rows, sem).wait()
            pltpu.sync_copy(rows, acc_sp.at[didx_h.at[jj]], add=True)

        def pair(t, _):
            j = 2 * t
            start(j + 1, rows_b, sem_b)
            finish(j, rows_a, sem_a)

            @pl.when(j + 2 < cpt2)
            def _():
                start(j + 2, rows_a, sem_a)

            finish(j + 1, rows_b, sem_b)
            return 0

        for h in range(2):
            base = w * CPT + h * cpt2
            if h == 0:
                pltpu.make_async_copy(srcm_hbm.at[pl.ds(base, cpt2), :],
                                      sidx_h, sem_a).wait()
                pltpu.make_async_copy(dstm_hbm.at[pl.ds(base, cpt2), :],
                                      didx_h, sem_b).wait()
            else:
                pltpu.sync_copy(srcm_hbm.at[pl.ds(base, cpt2), :], sidx_h)
                pltpu.sync_copy(dstm_hbm.at[pl.ds(base, cpt2), :], didx_h)
            start(0, rows_a, sem_a)
            lax.fori_loop(0, cpt2 // 2, pair, 0)
        plsc.subcore_barrier()

        def cout(k, _):
            r = s * rpt + k * coc
            pltpu.sync_copy(acc_sp.at[pl.ds(r, coc), :],
                            rows_a.at[pl.ds(0, coc), :])
            pltpu.sync_copy(rows_a.at[pl.ds(0, coc), :],
                            out_hbm.at[c, pl.ds(r, coc), :])
            return 0

        lax.fori_loop(0, nch, cout, 0)

    return pl.kernel(
        body,
        out_type=jax.ShapeDtypeStruct((NC, n, f), jnp.float32),
        mesh=mesh,
        scratch_types=[
            pltpu.VMEM_SHARED((n + coc, f), jnp.float32),
            pltpu.VMEM((CK, f), jnp.float32),
            pltpu.VMEM((CK, f), jnp.float32),
            pltpu.VMEM((cpt2, CK), jnp.int32),
            pltpu.VMEM((cpt2, CK), jnp.int32),
            pltpu.SemaphoreType.DMA,
            pltpu.SemaphoreType.DMA,
        ],
        compiler_params=pltpu.CompilerParams(use_tc_tiling_on_sc=tc_tiling),
    )


def _dinv_from(degp_ref):
    dp = degp_ref[...]
    deg = dp[:, 0] + dp[:, 1] + 1.0
    return lax.rsqrt(deg)


def _tc_xw_body(x_ref, w_ref, xw_ref):
    xw_ref[...] = jnp.dot(x_ref[...], w_ref[...],
                          preferred_element_type=jnp.float32)


def _tc1_body(xw_ref, degp_ref, y_ref):
    dinv = _dinv_from(degp_ref)
    y_ref[...] = xw_ref[...] * dinv[:, None]


def _tc2_body(a_ref, y1_ref, degp_ref, b1_ref, w2_ref, y2_ref):
    dinv = _dinv_from(degp_ref)
    a = a_ref[...]
    srow = a[0] + a[1] + y1_ref[...]
    h = jax.nn.relu(srow * dinv[:, None] + b1_ref[...])
    y2_ref[...] = jnp.dot(h, w2_ref[...],
                          preferred_element_type=jnp.float32) * dinv[:, None]


def _tc3_body(a_ref, y2_ref, degp_ref, b2_ref, o_ref):
    dinv = _dinv_from(degp_ref)
    a = a_ref[...]
    o = (a[0] + a[1] + y2_ref[...]) * dinv[:, None] + b2_ref[...]
    m = jnp.max(o, axis=1, keepdims=True)
    l = o - m
    o_ref[...] = l - jnp.log(jnp.sum(jnp.exp(l), axis=1, keepdims=True))


def kernel(x, edge_index, W1, b1, W2, b2):
    n, f_in = x.shape
    e = edge_index.shape[1]
    nhid = W1.shape[1]
    ncls = W2.shape[1]

    nw = NC * NS
    epw = e // nw
    pad_pt = CPT * CK - epw
    src = jnp.concatenate(
        [edge_index[0].reshape(nw, epw),
         jnp.broadcast_to(jnp.arange(pad_pt, dtype=jnp.int32) % n,
                          (nw, pad_pt))], axis=1).reshape(-1, CK)
    sink = n + (jnp.arange(pad_pt, dtype=jnp.int32) % 80)
    dst = jnp.concatenate(
        [edge_index[1].reshape(nw, epw),
         jnp.broadcast_to(sink, (nw, pad_pt))], axis=1).reshape(-1, CK)

    degp = _make_degree_kernel(n)(dst).reshape(NC, n).T

    blk = 1000
    grid = (n // blk,)
    xw = pl.pallas_call(
        _tc_xw_body,
        grid=grid,
        in_specs=[
            pl.BlockSpec((blk, f_in), lambda i: (i, 0)),
            pl.BlockSpec((f_in, nhid), lambda i: (0, 0)),
        ],
        out_specs=pl.BlockSpec((blk, nhid), lambda i: (i, 0)),
        out_shape=jax.ShapeDtypeStruct((n, nhid), jnp.float32),
    )(x, W1)
    y1 = pl.pallas_call(
        _tc1_body,
        grid=grid,
        in_specs=[
            pl.BlockSpec((blk, nhid), lambda i: (i, 0)),
            pl.BlockSpec((blk, NC), lambda i: (i, 0)),
        ],
        out_specs=pl.BlockSpec((blk, nhid), lambda i: (i, 0)),
        out_shape=jax.ShapeDtypeStruct((n, nhid), jnp.float32),
    )(xw, degp)

    a1 = _make_agg_kernel(n, nhid)(y1, src, dst)

    y2 = pl.pallas_call(
        _tc2_body,
        grid=grid,
        in_specs=[
            pl.BlockSpec((NC, blk, nhid), lambda i: (0, i, 0)),
            pl.BlockSpec((blk, nhid), lambda i: (i, 0)),
            pl.BlockSpec((blk, NC), lambda i: (i, 0)),
            pl.BlockSpec((1, nhid), lambda i: (0, 0)),
            pl.BlockSpec((nhid, ncls), lambda i: (0, 0)),
        ],
        out_specs=pl.BlockSpec((blk, ncls), lambda i: (i, 0)),
        out_shape=jax.ShapeDtypeStruct((n, ncls), jnp.float32),
    )(a1, y1, degp, b1.reshape(1, nhid), W2)

    a2 = _make_agg_kernel(n, ncls, tc_tiling=False)(y2, src, dst)

    out = pl.pallas_call(
        _tc3_body,
        grid=grid,
        in_specs=[
            pl.BlockSpec((NC, blk, ncls), lambda i: (0, i, 0)),
            pl.BlockSpec((blk, ncls), lambda i: (i, 0)),
            pl.BlockSpec((blk, NC), lambda i: (i, 0)),
            pl.BlockSpec((1, ncls), lambda i: (0, 0)),
        ],
        out_specs=pl.BlockSpec((blk, ncls), lambda i: (i, 0)),
        out_shape=jax.ShapeDtypeStruct((n, ncls), jnp.float32),
    )(a2, y2, degp, b2.reshape(1, ncls))

    return out


# speedup vs baseline: 1.0041x; 1.0041x over previous
import functools

import jax
import jax.numpy as jnp
from jax import lax
from jax.experimental import pallas as pl
from jax.experimental.pallas import tpu as pltpu
from jax.experimental.pallas import tpu_sc as plsc

NC = 2
NS = 16
CK = 128
CPT = 80


def _zero_f32(ref, rows, cols):
    z = jnp.zeros((16,), jnp.float32)

    def body(i, _):
        for g in range(cols // 16):
            ref[i, pl.ds(g * 16, 16)] = z
        return 0

    lax.fori_loop(0, rows, body, 0)


def _make_degree_kernel(n):
    npad = ((n + NS * 16 - 1) // (NS * 16)) * (NS * 16)
    rpt = npad // NS
    mesh = plsc.VectorSubcoreMesh(
        core_axis_name="c", subcore_axis_name="s", num_cores=NC,
        num_subcores=NS)

    def body(dstm_hbm, deg_out, deg_sp, zbuf, ones_v, idx_all):
        c = lax.axis_index("c")
        s = lax.axis_index("s")
        w = c * NS + s
        zv = jnp.zeros((16,), jnp.float32)
        ov = jnp.ones((16,), jnp.float32)

        def z16(i, _):
            zbuf[pl.ds(i * 16, 16)] = zv
            return 0

        lax.fori_loop(0, rpt // 16, z16, 0)
        for g in range(CK // 16):
            ones_v[pl.ds(g * 16, 16)] = ov
        pltpu.sync_copy(dstm_hbm.at[pl.ds(w * CPT, CPT), :], idx_all)
        pltpu.sync_copy(zbuf, deg_sp.at[pl.ds(s * rpt, rpt)])
        plsc.subcore_barrier()

        def chunk(j, _):
            pltpu.sync_copy(ones_v, deg_sp.at[idx_all.at[j]], add=True)
            return 0

        lax.fori_loop(0, CPT, chunk, 0)
        plsc.subcore_barrier()

        low = s * rpt
        size_last = n - rpt * (NS - 1)

        @pl.when(s < NS - 1)
        def _():
            pltpu.sync_copy(deg_sp.at[pl.ds(low, rpt)], zbuf)
            pltpu.sync_copy(zbuf, deg_out.at[pl.ds(c * n + low, rpt)])

        @pl.when(s == NS - 1)
        def _():
            pltpu.sync_copy(deg_sp.at[pl.ds(low, size_last)],
                            zbuf.at[pl.ds(0, size_last)])
            pltpu.sync_copy(zbuf.at[pl.ds(0, size_last)],
                            deg_out.at[pl.ds(c * n + low, size_last)])

    return pl.kernel(
        body,
        out_type=jax.ShapeDtypeStruct((NC * n,), jnp.float32),
        mesh=mesh,
        scratch_types=[
            pltpu.VMEM_SHARED((npad,), jnp.float32),
            pltpu.VMEM((rpt,), jnp.float32),
            pltpu.VMEM((CK,), jnp.float32),
            pltpu.VMEM((CPT, CK), jnp.int32),
        ],
    )


def _make_agg_kernel(n, f, tc_tiling=True):
    coc = 80
    rpt = ((n + NS * coc - 1) // (NS * coc)) * coc
    last = n - rpt * (NS - 1)
    assert rpt % coc == 0 and last % coc == 0 and last > 0
    mesh = plsc.VectorSubcoreMesh(
        core_axis_name="c", subcore_axis_name="s", num_cores=NC,
        num_subcores=NS)

    cpt2 = CPT // 2

    def body(y_hbm, srcm_hbm, dstm_hbm, out_hbm, acc_sp,
             rows_a, rows_b, sidx_h, didx_h, sem_a, sem_b):
        c = lax.axis_index("c")
        s = lax.axis_index("s")
        w = c * NS + s
        nch = jnp.where(s == NS - 1, last // coc, rpt // coc)
        nzch = jnp.where(s == NS - 1, (last + coc) // coc, rpt // coc)

        pltpu.async_copy(srcm_hbm.at[pl.ds(w * CPT, cpt2), :], sidx_h, sem_a)
        pltpu.async_copy(dstm_hbm.at[pl.ds(w * CPT, cpt2), :], didx_h, sem_b)

        _zero_f32(rows_a, coc, f)

        def zcp(k, _):
            pltpu.sync_copy(rows_a.at[pl.ds(0, coc), :],
                            acc_sp.at[pl.ds(s * rpt + k * coc, coc), :])
            return 0

        lax.fori_loop(0, nzch, zcp, 0)
        plsc.subcore_barrier()

        def start(jj, rows, sem):
            pltpu.async_copy(y_hbm.at[sidx_h.at[jj]], rows, sem)

        def finish(jj, rows, sem):
            pltpu.make_async_copy(y_hbm.at[sidx_h.at[jj]], rows, sem).wait()
            pltpu.sync_copy(rows, acc_sp.at[didx_h.at[jj]], add=True)

        def pair(t, _):
            j = 2 * t
            start(j + 1, rows_b, sem_b)
            finish(j, rows_a, sem_a)

            @pl.when(j + 2 < cpt2)
            def _():
                start(j + 2, rows_a, sem_a)

            finish(j + 1, rows_b, sem_b)
            return 0

        for h in range(2):
            base = w * CPT + h * cpt2
            if h == 0:
                pltpu.make_async_copy(srcm_hbm.at[pl.ds(base, cpt2), :],
                                      sidx_h, sem_a).wait()
                pltpu.make_async_copy(dstm_hbm.at[pl.ds(base, cpt2), :],
                                      didx_h, sem_b).wait()
            else:
                pltpu.sync_copy(srcm_hbm.at[pl.ds(base, cpt2), :], sidx_h)
                pltpu.sync_copy(dstm_hbm.at[pl.ds(base, cpt2), :], didx_h)
            start(0, rows_a, sem_a)
            lax.fori_loop(0, cpt2 // 2, pair, 0)
        plsc.subcore_barrier()

        def cout(k, _):
            r = s * rpt + k * coc
            pltpu.sync_copy(acc_sp.at[pl.ds(r, coc), :],
                            rows_a.at[pl.ds(0, coc), :])
            pltpu.sync_copy(rows_a.at[pl.ds(0, coc), :],
                            out_hbm.at[c, pl.ds(r, coc), :])
            return 0

        lax.fori_loop(0, nch, cout, 0)

    return pl.kernel(
        body,
        out_type=jax.ShapeDtypeStruct((NC, n, f), jnp.float32),
        mesh=mesh,
        scratch_types=[
            pltpu.VMEM_SHARED((n + coc, f), jnp.float32),
            pltpu.VMEM((CK, f), jnp.float32),
            pltpu.VMEM((CK, f), jnp.float32),
            pltpu.VMEM((cpt2, CK), jnp.int32),
            pltpu.VMEM((cpt2, CK), jnp.int32),
            pltpu.SemaphoreType.DMA,
            pltpu.SemaphoreType.DMA,
        ],
        compiler_params=pltpu.CompilerParams(use_tc_tiling_on_sc=tc_tiling),
    )


def _dinv_from(degp_ref):
    dp = degp_ref[...]
    deg = dp[:, 0] + dp[:, 1] + 1.0
    return lax.rsqrt(deg)


def _tc1_body(x_ref, w_ref, degp_ref, y_ref):
    dinv = _dinv_from(degp_ref)
    xw = jnp.dot(x_ref[...], w_ref[...], preferred_element_type=jnp.float32)
    y_ref[...] = xw * dinv[:, None]


def _tc2_body(a_ref, y1_ref, degp_ref, b1_ref, w2_ref, y2_ref):
    dinv = _dinv_from(degp_ref)
    a = a_ref[...]
    srow = a[0] + a[1] + y1_ref[...]
    h = jax.nn.relu(srow * dinv[:, None] + b1_ref[...])
    y2_ref[...] = jnp.dot(h, w2_ref[...],
                          preferred_element_type=jnp.float32) * dinv[:, None]


def _tc3_body(a_ref, y2_ref, degp_ref, b2_ref, o_ref):
    dinv = _dinv_from(degp_ref)
    a = a_ref[...]
    o = (a[0] + a[1] + y2_ref[...]) * dinv[:, None] + b2_ref[...]
    m = jnp.max(o, axis=1, keepdims=True)
    l = o - m
    o_ref[...] = l - jnp.log(jnp.sum(jnp.exp(l), axis=1, keepdims=True))


def kernel(x, edge_index, W1, b1, W2, b2):
    n, f_in = x.shape
    e = edge_index.shape[1]
    nhid = W1.shape[1]
    ncls = W2.shape[1]

    nw = NC * NS
    epw = e // nw
    pad_pt = CPT * CK - epw
    src = jnp.concatenate(
        [edge_index[0].reshape(nw, epw),
         jnp.broadcast_to(jnp.arange(pad_pt, dtype=jnp.int32) % n,
                          (nw, pad_pt))], axis=1).reshape(-1, CK)
    sink = n + (jnp.arange(pad_pt, dtype=jnp.int32) % 80)
    dst = jnp.concatenate(
        [edge_index[1].reshape(nw, epw),
         jnp.broadcast_to(sink, (nw, pad_pt))], axis=1).reshape(-1, CK)

    degp = _make_degree_kernel(n)(dst).reshape(NC, n).T

    blk = 1000
    grid = (n // blk,)
    y1 = pl.pallas_call(
        _tc1_body,
        grid=grid,
        in_specs=[
            pl.BlockSpec((blk, f_in), lambda i: (i, 0)),
            pl.BlockSpec((f_in, nhid), lambda i: (0, 0)),
            pl.BlockSpec((blk, NC), lambda i: (i, 0)),
        ],
        out_specs=pl.BlockSpec((blk, nhid), lambda i: (i, 0)),
        out_shape=jax.ShapeDtypeStruct((n, nhid), jnp.float32),
    )(x, W1, degp)

    a1 = _make_agg_kernel(n, nhid)(y1, src, dst)

    y2 = pl.pallas_call(
        _tc2_body,
        grid=grid,
        in_specs=[
            pl.BlockSpec((NC, blk, nhid), lambda i: (0, i, 0)),
            pl.BlockSpec((blk, nhid), lambda i: (i, 0)),
            pl.BlockSpec((blk, NC), lambda i: (i, 0)),
            pl.BlockSpec((1, nhid), lambda i: (0, 0)),
            pl.BlockSpec((nhid, ncls), lambda i: (0, 0)),
        ],
        out_specs=pl.BlockSpec((blk, ncls), lambda i: (i, 0)),
        out_shape=jax.ShapeDtypeStruct((n, ncls), jnp.float32),
    )(a1, y1, degp, b1.reshape(1, nhid), W2)

    a2 = _make_agg_kernel(n, ncls, tc_tiling=False)(y2, src, dst)

    out = pl.pallas_call(
        _tc3_body,
        grid=grid,
        in_specs=[
            pl.BlockSpec((NC, blk, ncls), lambda i: (0, i, 0)),
            pl.BlockSpec((blk, ncls), lambda i: (i, 0)),
            pl.BlockSpec((blk, NC), lambda i: (i, 0)),
            pl.BlockSpec((1, ncls), lambda i: (0, 0)),
        ],
        out_specs=pl.BlockSpec((blk, ncls), lambda i: (i, 0)),
        out_shape=jax.ShapeDtypeStruct((n, ncls), jnp.float32),
    )(a2, y2, degp, b2.reshape(1, ncls))

    return out
